# baseline (speedup 1.0000x reference)
import jax
import jax.numpy as jnp
from jax import lax
from jax.experimental import pallas as pl
from jax.experimental.pallas import tpu as pltpu
from jax.experimental.pallas import tpu_sc as plsc

VOCAB = 100000
EMBED = 64
CPW = 50
L = 16

_info = plsc.get_sparse_core_info()
_NC = _info.num_cores


def _extract_col64(emb_ref, lane_vec, out_row_ref):
    iota = lax.iota(jnp.int32, L)
    for k in range(4):
        out_row_ref[pl.ds(k * L, L)] = plsc.load_gather(
            emb_ref, [iota + k * L, lane_vec])


def _body(ent_hbm, w2c_t, wemb_t, cemb_t,
          out_l, out_r, out_cl, out_cr,
          ent_v, widx_v, emb_v, blk64_v, sem, sem2):
    wid = lax.axis_index("s") * _NC + lax.axis_index("c")
    side = wid & 1
    tp = wid >> 1

    pltpu.sync_copy(ent_hbm, ent_v.at[pl.ds(0, 4)])
    iota = lax.iota(jnp.int32, L)
    ent_vec = ent_v[...]
    zeros = jnp.zeros((L,), jnp.int32)

    @pl.when(tp <= 6)
    def _cand():
        e = jnp.sum(jnp.where(iota == side, ent_vec, 0))
        c0w = pl.multiple_of((e >> 7) << 7, 128)
        lw = zeros + (e & 127)
        row_start = pl.multiple_of(8 * tp, 8)
        pltpu.sync_copy(
            w2c_t.at[pl.ds(row_start, 8), pl.ds(c0w, 128)], widx_v)
        lanes = []
        copies = []
        for i in range(8):
            idv = plsc.load_gather(widx_v.at[i], [lw])
            rid = jnp.sum(jnp.where(iota == 0, idv, 0))
            rid = jnp.clip(rid, 0, VOCAB - 1)
            c0 = pl.multiple_of((rid >> 7) << 7, 128)
            lanes.append(zeros + (rid & 127))
            copies.append(pltpu.async_copy(
                cemb_t.at[:, pl.ds(c0, 128)], emb_v.at[i], sem))
        for i in range(8):
            copies[i].wait()
            _extract_col64(emb_v.at[i], lanes[i], blk64_v.at[i])
        out_start = pl.multiple_of(8 * tp, 8)

        @pl.when(side == 0)
        def _():
            pltpu.sync_copy(blk64_v, out_cl.at[pl.ds(out_start, 8), :])

        @pl.when(side == 1)
        def _():
            pltpu.sync_copy(blk64_v, out_cr.at[pl.ds(out_start, 8), :])

    @pl.when(tp == 15)
    def _word():
        e = jnp.sum(jnp.where(iota == 2 + side, ent_vec, 0))
        c0 = pl.multiple_of((e >> 7) << 7, 128)
        lw = zeros + (e & 127)
        pltpu.async_copy(wemb_t.at[:, pl.ds(c0, 128)], emb_v.at[0],
                         sem2).wait()
        _extract_col64(emb_v.at[0], lw, blk64_v.at[0])

        @pl.when(side == 0)
        def _():
            pltpu.sync_copy(blk64_v.at[pl.ds(0, 1), :], out_l)

        @pl.when(side == 1)
        def _():
            pltpu.sync_copy(blk64_v.at[pl.ds(0, 1), :], out_r)


_sc_call = pl.kernel(
    _body,
    out_type=(
        jax.ShapeDtypeStruct((1, EMBED), jnp.float32),
        jax.ShapeDtypeStruct((1, EMBED), jnp.float32),
        jax.ShapeDtypeStruct((CPW, EMBED), jnp.float32),
        jax.ShapeDtypeStruct((CPW, EMBED), jnp.float32),
    ),
    mesh=plsc.VectorSubcoreMesh(core_axis_name="c", subcore_axis_name="s"),
    compiler_params=pltpu.CompilerParams(use_tc_tiling_on_sc=True,
                                         needs_layout_passes=False,
                                         disable_bounds_checks=True,
                                         disable_semaphore_checks=True),
    scratch_types=[
        pltpu.VMEM((L,), jnp.int32),
        pltpu.VMEM((8, 128), jnp.int32),
        pltpu.VMEM((8, 64, 128), jnp.float32),
        pltpu.VMEM((8, EMBED), jnp.float32),
        pltpu.SemaphoreType.DMA,
        pltpu.SemaphoreType.DMA,
    ],
)


def kernel(entity, word2candidates, word_embeds, candidate_embeds):
    return _sc_call(
        entity, word2candidates.T, word_embeds.T, candidate_embeds.T)


# speedup vs baseline: 1.0011x; 1.0011x over previous
import jax
import jax.numpy as jnp
from jax import lax
from jax.experimental import pallas as pl
from jax.experimental.pallas import tpu as pltpu
from jax.experimental.pallas import tpu_sc as plsc

VOCAB = 100000
EMBED = 64
CPW = 50
L = 16

_info = plsc.get_sparse_core_info()
_NC = _info.num_cores


def _extract_col64(emb_ref, lane_vec, out_row_ref):
    iota = lax.iota(jnp.int32, L)
    for k in range(4):
        out_row_ref[pl.ds(k * L, L)] = plsc.load_gather(
            emb_ref, [iota + k * L, lane_vec])


def _body(ent_hbm, w2c_t, wemb_t, cemb_t,
          out_l, out_r, out_cl, out_cr,
          ent_v, widx_v, emb_v, blk64_v, sem, sem2):
    wid = lax.axis_index("s") * _NC + lax.axis_index("c")
    side = wid & 1
    tp = wid >> 1

    pltpu.sync_copy(ent_hbm, ent_v.at[pl.ds(0, 4)])
    iota = lax.iota(jnp.int32, L)
    ent_vec = ent_v[...]
    zeros = jnp.zeros((L,), jnp.int32)

    @pl.when(tp <= 6)
    def _cand():
        e = jnp.sum(jnp.where(iota == side, ent_vec, 0))
        c0w = pl.multiple_of((e >> 7) << 7, 128)
        lw = zeros + (e & 127)
        row_start = pl.multiple_of(8 * tp, 8)
        pltpu.sync_copy(
            w2c_t.at[pl.ds(row_start, 8), pl.ds(c0w, 128)], widx_v)
        lanes = []
        copies = []
        for i in range(8):
            idv = plsc.load_gather(widx_v.at[i], [lw])
            rid = jnp.sum(jnp.where(iota == 0, idv, 0))
            rid = jnp.clip(rid, 0, VOCAB - 1)
            c0 = pl.multiple_of((rid >> 7) << 7, 128)
            lanes.append(zeros + (rid & 127))
            copies.append(pltpu.async_copy(
                cemb_t.at[:, pl.ds(c0, 128)], emb_v.at[i], sem))
        for i in range(8):
            copies[i].wait()
            _extract_col64(emb_v.at[i], lanes[i], blk64_v.at[i])
        out_start = pl.multiple_of(8 * tp, 8)

        @pl.when(side == 0)
        def _():
            pltpu.sync_copy(blk64_v, out_cl.at[pl.ds(out_start, 8), :])

        @pl.when(side == 1)
        def _():
            pltpu.sync_copy(blk64_v, out_cr.at[pl.ds(out_start, 8), :])

    @pl.when(tp == 15)
    def _word():
        e = jnp.sum(jnp.where(iota == 2 + side, ent_vec, 0))
        c0 = pl.multiple_of((e >> 7) << 7, 128)
        lw = zeros + (e & 127)
        pltpu.async_copy(wemb_t.at[:, pl.ds(c0, 128)], emb_v.at[0],
                         sem2).wait()
        _extract_col64(emb_v.at[0], lw, blk64_v.at[0])

        @pl.when(side == 0)
        def _():
            pltpu.sync_copy(blk64_v.at[pl.ds(0, 1), :], out_l)

        @pl.when(side == 1)
        def _():
            pltpu.sync_copy(blk64_v.at[pl.ds(0, 1), :], out_r)


_sc_call = pl.kernel(
    _body,
    out_type=(
        jax.ShapeDtypeStruct((1, EMBED), jnp.float32),
        jax.ShapeDtypeStruct((1, EMBED), jnp.float32),
        jax.ShapeDtypeStruct((CPW, EMBED), jnp.float32),
        jax.ShapeDtypeStruct((CPW, EMBED), jnp.float32),
    ),
    mesh=plsc.VectorSubcoreMesh(core_axis_name="c", subcore_axis_name="s"),
    compiler_params=pltpu.CompilerParams(use_tc_tiling_on_sc=True,
                                         needs_layout_passes=False,
                                         disable_bounds_checks=True,
                                         disable_semaphore_checks=True,
                                         skip_device_barrier=True),
    scratch_types=[
        pltpu.VMEM((L,), jnp.int32),
        pltpu.VMEM((8, 128), jnp.int32),
        pltpu.VMEM((8, 64, 128), jnp.float32),
        pltpu.VMEM((8, EMBED), jnp.float32),
        pltpu.SemaphoreType.DMA,
        pltpu.SemaphoreType.DMA,
    ],
)


def kernel(entity, word2candidates, word_embeds, candidate_embeds):
    return _sc_call(
        entity, word2candidates.T, word_embeds.T, candidate_embeds.T)


# speedup vs baseline: 1.0339x; 1.0327x over previous
import jax
import jax.numpy as jnp
from jax import lax
from jax.experimental import pallas as pl
from jax.experimental.pallas import tpu as pltpu
from jax.experimental.pallas import tpu_sc as plsc

VOCAB = 100000
EMBED = 64
CPW = 50
L = 16

_info = plsc.get_sparse_core_info()
_NC = _info.num_cores


def _extract_col64(emb_ref, lane_vec, out_row_ref):
    iota = lax.iota(jnp.int32, L)
    for k in range(4):
        out_row_ref[pl.ds(k * L, L)] = plsc.load_gather(
            emb_ref, [iota + k * L, lane_vec])


def _body(ent_hbm, w2c_t, wemb_t, cemb_t,
          out_l, out_r, out_cl, out_cr,
          ent_v, widx_v, emb_v, blk64_v, sem, sem2):
    wid = lax.axis_index("s") * _NC + lax.axis_index("c")
    side = wid & 1
    tp = wid >> 1

    pltpu.sync_copy(ent_hbm, ent_v.at[pl.ds(0, 4)])
    iota = lax.iota(jnp.int32, L)
    ent_vec = ent_v[...]
    zeros = jnp.zeros((L,), jnp.int32)

    @pl.when(tp <= 6)
    def _cand():
        e = jnp.sum(jnp.where(iota == side, ent_vec, 0))
        c0w = pl.multiple_of((e >> 7) << 7, 128)
        lw = zeros + (e & 127)
        row_start = pl.multiple_of(8 * tp, 8)
        pltpu.sync_copy(
            w2c_t.at[pl.ds(row_start, 8), pl.ds(c0w, 128)], widx_v)
        lanes = []
        copies = []
        for i in range(8):
            idv = plsc.load_gather(widx_v.at[i], [lw])
            rid = jnp.sum(jnp.where(iota == 0, idv, 0))
            rid = jnp.clip(rid, 0, VOCAB - 1)
            c0 = pl.multiple_of((rid >> 7) << 7, 128)
            lanes.append(zeros + (rid & 127))
            copies.append(pltpu.async_copy(
                cemb_t.at[:, pl.ds(c0, 128)], emb_v.at[i], sem))
        for i in range(8):
            copies[i].wait()
            _extract_col64(emb_v.at[i], lanes[i], blk64_v.at[i])
        out_start = pl.multiple_of(8 * tp, 8)

        @pl.when(side == 0)
        def _():
            pltpu.sync_copy(blk64_v, out_cl.at[pl.ds(out_start, 8), :])

        @pl.when(side == 1)
        def _():
            pltpu.sync_copy(blk64_v, out_cr.at[pl.ds(out_start, 8), :])

    @pl.when(tp == 15)
    def _word():
        e = jnp.sum(jnp.where(iota == 2 + side, ent_vec, 0))
        c0 = pl.multiple_of((e >> 7) << 7, 128)
        lw = zeros + (e & 127)
        pltpu.async_copy(wemb_t.at[:, pl.ds(c0, 128)], emb_v.at[0],
                         sem2).wait()
        _extract_col64(emb_v.at[0], lw, blk64_v.at[0])

        @pl.when(side == 0)
        def _():
            pltpu.sync_copy(blk64_v.at[pl.ds(0, 1), :], out_l)

        @pl.when(side == 1)
        def _():
            pltpu.sync_copy(blk64_v.at[pl.ds(0, 1), :], out_r)


_sc_call = pl.kernel(
    _body,
    out_type=(
        jax.ShapeDtypeStruct((1, EMBED), jnp.float32),
        jax.ShapeDtypeStruct((1, EMBED), jnp.float32),
        jax.ShapeDtypeStruct((CPW, EMBED), jnp.float32),
        jax.ShapeDtypeStruct((CPW, EMBED), jnp.float32),
    ),
    mesh=plsc.VectorSubcoreMesh(core_axis_name="c", subcore_axis_name="s"),
    compiler_params=pltpu.CompilerParams(use_tc_tiling_on_sc=True,
                                         needs_layout_passes=False,
                                         disable_bounds_checks=True,
                                         disable_semaphore_checks=True),
    scratch_types=[
        pltpu.VMEM((L,), jnp.int32),
        pltpu.VMEM((8, 128), jnp.int32),
        pltpu.VMEM((8, 64, 128), jnp.float32),
        pltpu.VMEM((8, EMBED), jnp.float32),
        pltpu.SemaphoreType.DMA,
        pltpu.SemaphoreType.DMA,
    ],
)


def kernel(entity, word2candidates, word_embeds, candidate_embeds):
    return _sc_call(
        entity, word2candidates.T, word_embeds.T, candidate_embeds.T)
